# Initial kernel scaffold; baseline (speedup 1.0000x reference)
import jax
import jax.numpy as jnp
from jax.experimental import pallas as pl


def kernel(hidden_states, top_indices, Wq, bq, Wk, bk, Wv, bv, Wo, bo, Wg, bg, Wr, br):
    raise NotImplementedError("write your pallas kernel here")


# speedup vs baseline: 1.1789x; 1.1789x over previous
import math

import jax
import jax.numpy as jnp
from jax.experimental import pallas as pl
from jax.experimental.pallas import tpu as pltpu

_B, _T, _C, _H, _HS, _BS, _NB, _NS, _W = 1, 2048, 768, 12, 64, 64, 32, 8, 128
_TQ = 256
_SCALE = 1.0 / math.sqrt(_HS)
_NPROJ = 3 * _C + 128


def _matmul_kernel(x_ref, w_ref, b_ref, o_ref):
    o_ref[...] = (
        jnp.dot(x_ref[...], w_ref[...], preferred_element_type=jnp.float32)
        + b_ref[...]
    )


def _matmul(x, w, b, tile_m=256):
    m, k = x.shape
    n = w.shape[1]
    return pl.pallas_call(
        _matmul_kernel,
        grid=(m // tile_m,),
        in_specs=[
            pl.BlockSpec((tile_m, k), lambda i: (i, 0)),
            pl.BlockSpec((k, n), lambda i: (0, 0)),
            pl.BlockSpec((1, n), lambda i: (0, 0)),
        ],
        out_specs=pl.BlockSpec((tile_m, n), lambda i: (i, 0)),
        out_shape=jax.ShapeDtypeStruct((m, n), jnp.float32),
    )(x, w, b)


def _attn_kernel(q_ref, k_ref, v_ref, aux_ref, idx_ref, o_ref, ent_ref):
    h = pl.program_id(0)
    qt = pl.program_id(1)
    q = q_ref[0]
    k = k_ref[0]
    v = v_ref[0]
    dn = (((1,), (1,)), ((), ()))
    s = jax.lax.dot_general(q, k, dn, preferred_element_type=jnp.float32) * _SCALE
    q0 = qt * _TQ
    row = jax.lax.broadcasted_iota(jnp.int32, (_TQ, _T), 0) + q0
    col = jax.lax.broadcasted_iota(jnp.int32, (_TQ, _T), 1)

    s3 = s.reshape(_TQ, _NB, _BS)
    m3 = jnp.max(s3, axis=-1, keepdims=True)
    e3 = jnp.exp(s3 - m3)
    p3 = e3 / jnp.sum(e3, axis=-1, keepdims=True)
    ent_ref[0] = -jnp.sum(p3 * jnp.log(p3 + 1e-9), axis=-1)

    sm = jnp.where((col <= row) & (col >= row - _W), s, -1e9)
    mx = jnp.max(sm, axis=-1, keepdims=True)
    ex = jnp.exp(sm - mx)
    p_sl = ex / jnp.sum(ex, axis=-1, keepdims=True)
    attn_sl = jnp.dot(p_sl, v, preferred_element_type=jnp.float32)

    sel_k = jnp.concatenate(
        [k_ref[0, pl.ds(idx_ref[h, sl] * _BS, _BS), :] for sl in range(_NS)], axis=0
    )
    sel_v = jnp.concatenate(
        [v_ref[0, pl.ds(idx_ref[h, sl] * _BS, _BS), :] for sl in range(_NS)], axis=0
    )
    ss = jax.lax.dot_general(q, sel_k, dn, preferred_element_type=jnp.float32) * _SCALE
    rowc = jax.lax.broadcasted_iota(jnp.int32, (_TQ, _NS * _BS), 0) + q0
    cpos = jax.lax.broadcasted_iota(jnp.int32, (_TQ, _NS * _BS), 1)
    ssm = jnp.where(cpos <= rowc, ss, -1e9)
    mxs = jnp.max(ssm, axis=-1, keepdims=True)
    exs = jnp.exp(ssm - mxs)
    p_sel = exs / jnp.sum(exs, axis=-1, keepdims=True)
    attn_sel = jnp.dot(p_sel, sel_v, preferred_element_type=jnp.float32)

    kc = jnp.mean(k.reshape(_NB, _BS, _HS), axis=1)
    vc = jnp.mean(v.reshape(_NB, _BS, _HS), axis=1)
    cs = jax.lax.dot_general(q, kc, dn, preferred_element_type=jnp.float32) * _SCALE
    rowb = jax.lax.broadcasted_iota(jnp.int32, (_TQ, _NB), 0) + q0
    colb = jax.lax.broadcasted_iota(jnp.int32, (_TQ, _NB), 1)
    csm = jnp.where(colb <= rowb, cs, -1e9)
    mxc = jnp.max(csm, axis=-1, keepdims=True)
    exc = jnp.exp(csm - mxc)
    p_cmp = exc / jnp.sum(exc, axis=-1, keepdims=True)
    attn_cmp = jnp.dot(p_cmp, vc, preferred_element_type=jnp.float32)

    gl = aux_ref[...][:, 32:35]
    gm = jnp.max(gl, axis=-1, keepdims=True)
    ge = jnp.exp(gl - gm)
    g = ge / jnp.sum(ge, axis=-1, keepdims=True)
    o_ref[0] = (
        g[:, 0:1] * attn_sl + g[:, 1:2] * attn_sel + g[:, 2:3] * attn_cmp
    )


def kernel(hidden_states, top_indices, Wq, bq, Wk, bk, Wv, bv, Wo, bo, Wg, bg, Wr, br):
    x = hidden_states.reshape(_T, _C)
    pad = _NPROJ - 3 * _C - _NB - 3
    w_all = jnp.concatenate(
        [Wq.T, Wk.T, Wv.T, Wr.T, Wg.T, jnp.zeros((_C, pad), jnp.float32)], axis=1
    )
    b_all = jnp.concatenate(
        [bq, bk, bv, br, bg, jnp.zeros((pad,), jnp.float32)]
    ).reshape(1, _NPROJ)
    proj = _matmul(x, w_all, b_all)

    q3 = proj[:, 0 : _C].reshape(_T, _H, _HS).transpose(1, 0, 2)
    k3 = proj[:, _C : 2 * _C].reshape(_T, _H, _HS).transpose(1, 0, 2)
    v3 = proj[:, 2 * _C : 3 * _C].reshape(_T, _H, _HS).transpose(1, 0, 2)
    aux = proj[:, 3 * _C :]
    router_logits = proj[:, 3 * _C : 3 * _C + _NB].reshape(_B, _T, _NB)
    idx = top_indices.reshape(_H, _NS).astype(jnp.int32)

    o3, ent = pl.pallas_call(
        _attn_kernel,
        grid=(_H, _T // _TQ),
        in_specs=[
            pl.BlockSpec((1, _TQ, _HS), lambda h, i: (h, i, 0)),
            pl.BlockSpec((1, _T, _HS), lambda h, i: (h, 0, 0)),
            pl.BlockSpec((1, _T, _HS), lambda h, i: (h, 0, 0)),
            pl.BlockSpec((_TQ, 128), lambda h, i: (i, 0)),
            pl.BlockSpec(memory_space=pltpu.SMEM),
        ],
        out_specs=[
            pl.BlockSpec((1, _TQ, _HS), lambda h, i: (h, i, 0)),
            pl.BlockSpec((1, _TQ, _NB), lambda h, i: (h, i, 0)),
        ],
        out_shape=[
            jax.ShapeDtypeStruct((_H, _T, _HS), jnp.float32),
            jax.ShapeDtypeStruct((_H, _T, _NB), jnp.float32),
        ],
    )(q3, k3, v3, aux, idx)

    attn_btc = o3.transpose(1, 0, 2).reshape(_T, _C)
    out = _matmul(attn_btc, Wo.T, bo.reshape(1, _C))
    return out.reshape(_B, _T, _C), router_logits, ent[None]


# speedup vs baseline: 1.2412x; 1.0528x over previous
import math

import jax
import jax.numpy as jnp
from jax.experimental import pallas as pl
from jax.experimental.pallas import tpu as pltpu

_B, _T, _C, _H, _HS, _BS, _NB, _NS, _W = 1, 2048, 768, 12, 64, 64, 32, 8, 128
_TQ = 256
_SCALE = 1.0 / math.sqrt(_HS)
_NPROJ = 3 * _C + 128


def _matmul_kernel(x_ref, w_ref, b_ref, o_ref):
    o_ref[...] = (
        jnp.dot(x_ref[...], w_ref[...], preferred_element_type=jnp.float32)
        + b_ref[...]
    )


def _matmul(x, w, b, tile_m=256):
    m, k = x.shape
    n = w.shape[1]
    return pl.pallas_call(
        _matmul_kernel,
        grid=(m // tile_m,),
        in_specs=[
            pl.BlockSpec((tile_m, k), lambda i: (i, 0)),
            pl.BlockSpec((k, n), lambda i: (0, 0)),
            pl.BlockSpec((1, n), lambda i: (0, 0)),
        ],
        out_specs=pl.BlockSpec((tile_m, n), lambda i: (i, 0)),
        out_shape=jax.ShapeDtypeStruct((m, n), jnp.float32),
    )(x, w, b)


def _attn_kernel(q_ref, k_ref, v_ref, aux_ref, idx_ref, o_ref, ent_ref):
    h = pl.program_id(0)
    qt = pl.program_id(1)
    q = q_ref[0]
    k = k_ref[0]
    v = v_ref[0]
    dn = (((1,), (1,)), ((), ()))
    s = jax.lax.dot_general(q, k, dn, preferred_element_type=jnp.float32) * _SCALE
    q0 = qt * _TQ

    s3 = s.reshape(_TQ, _NB, _BS)
    m3 = jnp.max(s3, axis=-1, keepdims=True)
    d3 = s3 - m3
    e3 = jnp.exp(d3)
    z = jnp.sum(e3, axis=-1)
    u = jnp.sum(e3 * d3, axis=-1)
    ent_ref[0] = jnp.log(z) - u / z

    _BAND = _TQ + _W
    koff = jnp.maximum(q0 - _W, 0)
    k_band = k_ref[0, pl.ds(koff, _BAND), :]
    v_band = v_ref[0, pl.ds(koff, _BAND), :]
    sb = jax.lax.dot_general(q, k_band, dn, preferred_element_type=jnp.float32) * _SCALE
    rowb = jax.lax.broadcasted_iota(jnp.int32, (_TQ, _BAND), 0) + q0
    colb = jax.lax.broadcasted_iota(jnp.int32, (_TQ, _BAND), 1) + koff
    sm = jnp.where((colb <= rowb) & (colb >= rowb - _W), sb, -1e9)
    mx = jnp.max(sm, axis=-1, keepdims=True)
    ex = jnp.exp(sm - mx)
    p_sl = ex / jnp.sum(ex, axis=-1, keepdims=True)
    attn_sl = jnp.dot(p_sl, v_band, preferred_element_type=jnp.float32)

    sel_k = jnp.concatenate(
        [k_ref[0, pl.ds(idx_ref[h, sl] * _BS, _BS), :] for sl in range(_NS)], axis=0
    )
    sel_v = jnp.concatenate(
        [v_ref[0, pl.ds(idx_ref[h, sl] * _BS, _BS), :] for sl in range(_NS)], axis=0
    )
    ss = jax.lax.dot_general(q, sel_k, dn, preferred_element_type=jnp.float32) * _SCALE
    rowc = jax.lax.broadcasted_iota(jnp.int32, (_TQ, _NS * _BS), 0) + q0
    cpos = jax.lax.broadcasted_iota(jnp.int32, (_TQ, _NS * _BS), 1)
    ssm = jnp.where(cpos <= rowc, ss, -1e9)
    mxs = jnp.max(ssm, axis=-1, keepdims=True)
    exs = jnp.exp(ssm - mxs)
    p_sel = exs / jnp.sum(exs, axis=-1, keepdims=True)
    attn_sel = jnp.dot(p_sel, sel_v, preferred_element_type=jnp.float32)

    kc = jnp.mean(k.reshape(_NB, _BS, _HS), axis=1)
    vc = jnp.mean(v.reshape(_NB, _BS, _HS), axis=1)
    cs = jax.lax.dot_general(q, kc, dn, preferred_element_type=jnp.float32) * _SCALE
    rowb = jax.lax.broadcasted_iota(jnp.int32, (_TQ, _NB), 0) + q0
    colb = jax.lax.broadcasted_iota(jnp.int32, (_TQ, _NB), 1)
    csm = jnp.where(colb <= rowb, cs, -1e9)
    mxc = jnp.max(csm, axis=-1, keepdims=True)
    exc = jnp.exp(csm - mxc)
    p_cmp = exc / jnp.sum(exc, axis=-1, keepdims=True)
    attn_cmp = jnp.dot(p_cmp, vc, preferred_element_type=jnp.float32)

    gl = aux_ref[...][:, 32:35]
    gm = jnp.max(gl, axis=-1, keepdims=True)
    ge = jnp.exp(gl - gm)
    g = ge / jnp.sum(ge, axis=-1, keepdims=True)
    o_ref[0] = (
        g[:, 0:1] * attn_sl + g[:, 1:2] * attn_sel + g[:, 2:3] * attn_cmp
    )


def kernel(hidden_states, top_indices, Wq, bq, Wk, bk, Wv, bv, Wo, bo, Wg, bg, Wr, br):
    x = hidden_states.reshape(_T, _C)
    pad = _NPROJ - 3 * _C - _NB - 3
    w_all = jnp.concatenate(
        [Wq.T, Wk.T, Wv.T, Wr.T, Wg.T, jnp.zeros((_C, pad), jnp.float32)], axis=1
    )
    b_all = jnp.concatenate(
        [bq, bk, bv, br, bg, jnp.zeros((pad,), jnp.float32)]
    ).reshape(1, _NPROJ)
    proj = _matmul(x, w_all, b_all)

    q3 = proj[:, 0 : _C].reshape(_T, _H, _HS).transpose(1, 0, 2)
    k3 = proj[:, _C : 2 * _C].reshape(_T, _H, _HS).transpose(1, 0, 2)
    v3 = proj[:, 2 * _C : 3 * _C].reshape(_T, _H, _HS).transpose(1, 0, 2)
    aux = proj[:, 3 * _C :]
    router_logits = proj[:, 3 * _C : 3 * _C + _NB].reshape(_B, _T, _NB)
    idx = top_indices.reshape(_H, _NS).astype(jnp.int32)

    o3, ent = pl.pallas_call(
        _attn_kernel,
        grid=(_H, _T // _TQ),
        in_specs=[
            pl.BlockSpec((1, _TQ, _HS), lambda h, i: (h, i, 0)),
            pl.BlockSpec((1, _T, _HS), lambda h, i: (h, 0, 0)),
            pl.BlockSpec((1, _T, _HS), lambda h, i: (h, 0, 0)),
            pl.BlockSpec((_TQ, 128), lambda h, i: (i, 0)),
            pl.BlockSpec(memory_space=pltpu.SMEM),
        ],
        out_specs=[
            pl.BlockSpec((1, _TQ, _HS), lambda h, i: (h, i, 0)),
            pl.BlockSpec((1, _TQ, _NB), lambda h, i: (h, i, 0)),
        ],
        out_shape=[
            jax.ShapeDtypeStruct((_H, _T, _HS), jnp.float32),
            jax.ShapeDtypeStruct((_H, _T, _NB), jnp.float32),
        ],
    )(q3, k3, v3, aux, idx)

    attn_btc = o3.transpose(1, 0, 2).reshape(_T, _C)
    out = _matmul(attn_btc, Wo.T, bo.reshape(1, _C))
    return out.reshape(_B, _T, _C), router_logits, ent[None]


# speedup vs baseline: 2.6387x; 2.1260x over previous
import math

import jax
import jax.numpy as jnp
from jax.experimental import pallas as pl
from jax.experimental.pallas import tpu as pltpu

_B, _T, _C, _H, _HS, _BS, _NB, _NS, _W = 1, 2048, 768, 12, 64, 64, 32, 8, 128
_TQ = 256
_SCALE = 1.0 / math.sqrt(_HS)
_NPROJ = 3 * _C + 128


def _matmul_kernel(x_ref, w_ref, b_ref, o_ref):
    o_ref[...] = (
        jnp.dot(x_ref[...], w_ref[...], preferred_element_type=jnp.float32)
        + b_ref[...]
    )


def _matmul(x, w, b, tile_m=256):
    m, k = x.shape
    n = w.shape[1]
    return pl.pallas_call(
        _matmul_kernel,
        grid=(m // tile_m,),
        in_specs=[
            pl.BlockSpec((tile_m, k), lambda i: (i, 0)),
            pl.BlockSpec((k, n), lambda i: (0, 0)),
            pl.BlockSpec((1, n), lambda i: (0, 0)),
        ],
        out_specs=pl.BlockSpec((tile_m, n), lambda i: (i, 0)),
        out_shape=jax.ShapeDtypeStruct((m, n), jnp.float32),
    )(x, w, b)


def _attn_kernel(q_ref, k_ref, v_ref, aux_ref, g_ref, idx_ref, o_ref, ent_ref):
    h = pl.program_id(0)
    qt = pl.program_id(1)
    q = q_ref[0]
    k = k_ref[0]
    v = v_ref[0]
    gind = g_ref[...]
    dn = (((1,), (1,)), ((), ()))
    dn0 = (((0,), (0,)), ((), ()))
    s = jax.lax.dot_general(q, k, dn, preferred_element_type=jnp.float32) * _SCALE
    q0 = qt * _TQ

    e = jnp.exp(s)
    es = e * s
    z = jnp.dot(e, gind, preferred_element_type=jnp.float32)
    u = jnp.dot(es, gind, preferred_element_type=jnp.float32)
    ent_ref[0] = jnp.log(z) - u / z

    _BAND = _TQ + _W
    koff = jnp.maximum(q0 - _W, 0)
    k_band = k_ref[0, pl.ds(koff, _BAND), :]
    v_band = v_ref[0, pl.ds(koff, _BAND), :]
    sb = jax.lax.dot_general(q, k_band, dn, preferred_element_type=jnp.float32) * _SCALE
    rowb = jax.lax.broadcasted_iota(jnp.int32, (_TQ, _BAND), 0) + q0
    colb = jax.lax.broadcasted_iota(jnp.int32, (_TQ, _BAND), 1) + koff
    eb = jnp.where((colb <= rowb) & (colb >= rowb - _W), jnp.exp(sb), 0.0)
    zb = jnp.sum(eb, axis=-1, keepdims=True)
    attn_sl = jnp.dot(eb, v_band, preferred_element_type=jnp.float32) / zb

    sel_k = jnp.concatenate(
        [k_ref[0, pl.ds(idx_ref[h, sl] * _BS, _BS), :] for sl in range(_NS)], axis=0
    )
    sel_v = jnp.concatenate(
        [v_ref[0, pl.ds(idx_ref[h, sl] * _BS, _BS), :] for sl in range(_NS)], axis=0
    )
    ss = jax.lax.dot_general(q, sel_k, dn, preferred_element_type=jnp.float32) * _SCALE
    rowc = jax.lax.broadcasted_iota(jnp.int32, (_TQ, _NS * _BS), 0) + q0
    cpos = jax.lax.broadcasted_iota(jnp.int32, (_TQ, _NS * _BS), 1)
    es2 = jnp.where(cpos <= rowc, jnp.exp(ss), 0.0)
    zs = jnp.sum(es2, axis=-1, keepdims=True)
    attn_sel = jnp.dot(es2, sel_v, preferred_element_type=jnp.float32) / zs

    kc = jax.lax.dot_general(gind, k, dn0, preferred_element_type=jnp.float32) * (
        1.0 / _BS
    )
    vc = jax.lax.dot_general(gind, v, dn0, preferred_element_type=jnp.float32) * (
        1.0 / _BS
    )
    cs = jax.lax.dot_general(q, kc, dn, preferred_element_type=jnp.float32) * _SCALE
    rown = jax.lax.broadcasted_iota(jnp.int32, (_TQ, _NB), 0) + q0
    coln = jax.lax.broadcasted_iota(jnp.int32, (_TQ, _NB), 1)
    ec = jnp.where(coln <= rown, jnp.exp(cs), 0.0)
    zc = jnp.sum(ec, axis=-1, keepdims=True)
    attn_cmp = jnp.dot(ec, vc, preferred_element_type=jnp.float32) / zc

    gl = aux_ref[...][:, 32:35]
    gm = jnp.max(gl, axis=-1, keepdims=True)
    ge = jnp.exp(gl - gm)
    g = ge / jnp.sum(ge, axis=-1, keepdims=True)
    o_ref[0] = (
        g[:, 0:1] * attn_sl + g[:, 1:2] * attn_sel + g[:, 2:3] * attn_cmp
    )


def kernel(hidden_states, top_indices, Wq, bq, Wk, bk, Wv, bv, Wo, bo, Wg, bg, Wr, br):
    x = hidden_states.reshape(_T, _C)
    pad = _NPROJ - 3 * _C - _NB - 3
    w_all = jnp.concatenate(
        [Wq.T, Wk.T, Wv.T, Wr.T, Wg.T, jnp.zeros((_C, pad), jnp.float32)], axis=1
    )
    b_all = jnp.concatenate(
        [bq, bk, bv, br, bg, jnp.zeros((pad,), jnp.float32)]
    ).reshape(1, _NPROJ)
    proj = _matmul(x, w_all, b_all)

    q3 = proj[:, 0 : _C].reshape(_T, _H, _HS).transpose(1, 0, 2)
    k3 = proj[:, _C : 2 * _C].reshape(_T, _H, _HS).transpose(1, 0, 2)
    v3 = proj[:, 2 * _C : 3 * _C].reshape(_T, _H, _HS).transpose(1, 0, 2)
    aux = proj[:, 3 * _C :]
    router_logits = proj[:, 3 * _C : 3 * _C + _NB].reshape(_B, _T, _NB)
    idx = top_indices.reshape(_H, _NS).astype(jnp.int32)
    gind = (
        jnp.arange(_T, dtype=jnp.int32)[:, None] // _BS
        == jnp.arange(_NB, dtype=jnp.int32)[None, :]
    ).astype(jnp.float32)

    o3, ent = pl.pallas_call(
        _attn_kernel,
        grid=(_H, _T // _TQ),
        in_specs=[
            pl.BlockSpec((1, _TQ, _HS), lambda h, i: (h, i, 0)),
            pl.BlockSpec((1, _T, _HS), lambda h, i: (h, 0, 0)),
            pl.BlockSpec((1, _T, _HS), lambda h, i: (h, 0, 0)),
            pl.BlockSpec((_TQ, 128), lambda h, i: (i, 0)),
            pl.BlockSpec((_T, _NB), lambda h, i: (0, 0)),
            pl.BlockSpec(memory_space=pltpu.SMEM),
        ],
        out_specs=[
            pl.BlockSpec((1, _TQ, _HS), lambda h, i: (h, i, 0)),
            pl.BlockSpec((1, _TQ, _NB), lambda h, i: (h, i, 0)),
        ],
        out_shape=[
            jax.ShapeDtypeStruct((_H, _T, _HS), jnp.float32),
            jax.ShapeDtypeStruct((_H, _T, _NB), jnp.float32),
        ],
    )(q3, k3, v3, aux, gind, idx)

    attn_btc = o3.transpose(1, 0, 2).reshape(_T, _C)
    out = _matmul(attn_btc, Wo.T, bo.reshape(1, _C))
    return out.reshape(_B, _T, _C), router_logits, ent[None]
